# baseline (speedup 1.0000x reference)
import functools

import jax
import jax.numpy as jnp
from jax import lax
from jax.experimental import pallas as pl
from jax.experimental.pallas import tpu as pltpu
from jax.experimental.pallas import tpu_sc as plsc

N = 10000
E = 320000
D = 128

NC = 2
NS = 16
NW = NC * NS
E_PER_W = E // NW
CHUNK = 80
N_CHUNKS = E_PER_W // CHUNK
RING = 3
AHEAD = 2
N_MAIN = N_CHUNKS - (N_CHUNKS % RING)
N_OUTER = N_MAIN // RING
N_PAD = 10240
ROWS_PER_TILE = N_PAD // NS


def _sc_aggregate(xb, src, dst, zeros_tile):
    mesh = plsc.VectorSubcoreMesh(core_axis_name="c", subcore_axis_name="s")

    @functools.partial(
        pl.kernel,
        out_type=jax.ShapeDtypeStruct((NC, N_PAD, D), jnp.float32),
        mesh=mesh,
        scratch_types=[
            pltpu.VMEM_SHARED((N_PAD, D), jnp.float32),
            pltpu.VMEM((E_PER_W,), jnp.int32),
            pltpu.VMEM((RING, CHUNK), jnp.int32),
            pltpu.VMEM((RING, CHUNK, D), jnp.float32),
            pltpu.SemaphoreType.DMA((RING,)),
            pltpu.SemaphoreType.DMA((RING,)),
            pltpu.SemaphoreType.DMA((RING,)),
        ],
    )
    def k(x_hbm, src_hbm, dst_hbm, z_hbm, out_hbm,
          acc, src_v, dst_v, rows_v, gsem, ssem, dsem):
        c = lax.axis_index("c")
        s = lax.axis_index("s")
        wid = s * NC + c
        base = wid * E_PER_W

        def goff(g):
            return pl.multiple_of(g * CHUNK, 8)

        def gather_start(g, b):
            idx = src_v.at[pl.ds(goff(g), CHUNK)]
            pltpu.async_copy(x_hbm.at[idx], rows_v.at[b], gsem.at[b])

        def gather_wait(b):
            idx = src_v.at[pl.ds(0, CHUNK)]
            pltpu.make_async_copy(x_hbm.at[idx], rows_v.at[b], gsem.at[b]).wait()

        def dst_start(g, b):
            pltpu.async_copy(dst_hbm.at[pl.ds(base + goff(g), CHUNK)],
                             dst_v.at[b], dsem.at[b])

        def dst_wait(b):
            pltpu.make_async_copy(dst_hbm.at[pl.ds(base, CHUNK)],
                                  dst_v.at[b], dsem.at[b]).wait()

        def scatter_start(b):
            pltpu.async_copy(rows_v.at[b], acc.at[dst_v.at[b]], ssem.at[b],
                             add=True)

        def scatter_wait(b):
            pltpu.make_async_copy(rows_v.at[b], acc.at[dst_v.at[b]],
                                  ssem.at[b]).wait()

        pltpu.sync_copy(src_hbm.at[pl.ds(base, E_PER_W)], src_v)
        for g in range(AHEAD):
            dst_start(g, g)
            gather_start(g, g)
        pltpu.sync_copy(z_hbm, acc.at[pl.ds(s * ROWS_PER_TILE, ROWS_PER_TILE)])
        plsc.subcore_barrier()

        def step(g, b):
            b2 = (b + AHEAD) % RING
            gather_wait(b)
            dst_wait(b)
            scatter_start(b)

            @pl.when(g >= RING - AHEAD)
            def _():
                scatter_wait(b2)

            @pl.when(g + AHEAD < N_CHUNKS)
            def _():
                dst_start(g + AHEAD, b2)
                gather_start(g + AHEAD, b2)

        def body(i, _):
            for b in range(RING):
                step(i * RING + b, b)
            return ()

        lax.fori_loop(0, N_OUTER, body, ())

        for g in range(N_MAIN, N_CHUNKS):
            step(g, g % RING)

        for g in range(N_CHUNKS - (RING - AHEAD), N_CHUNKS):
            scatter_wait(g % RING)

        plsc.subcore_barrier()

        pltpu.sync_copy(
            acc.at[pl.ds(s * ROWS_PER_TILE, ROWS_PER_TILE)],
            out_hbm.at[c, pl.ds(s * ROWS_PER_TILE, ROWS_PER_TILE)],
        )

    return k(xb, src, dst, zeros_tile)


def _tc_out1(x, W1, b):
    BLK = 400

    def body(x_ref, w1_ref, b_ref, o_ref):
        dn = (((1,), (1,)), ((), ()))
        o_ref[...] = lax.dot_general(
            x_ref[...], w1_ref[...], dn,
            preferred_element_type=jnp.float32) + b_ref[...]

    return pl.pallas_call(
        body,
        grid=(N // BLK,),
        in_specs=[
            pl.BlockSpec((BLK, D), lambda i: (i, 0)),
            pl.BlockSpec((D, D), lambda i: (0, 0)),
            pl.BlockSpec((1, D), lambda i: (0, 0)),
        ],
        out_specs=pl.BlockSpec((BLK, D), lambda i: (i, 0)),
        out_shape=jax.ShapeDtypeStruct((N, D), jnp.float32),
    )(x, W1, b)


def _tc_combine(out1, partials, W2):
    BLK = 400

    def body(o1_ref, p_ref, w2_ref, o_ref):
        dn = (((1,), (1,)), ((), ()))
        agg = p_ref[0] + p_ref[1]
        o_ref[...] = o1_ref[...] + lax.dot_general(
            agg, w2_ref[...], dn, preferred_element_type=jnp.float32)

    return pl.pallas_call(
        body,
        grid=(N // BLK,),
        in_specs=[
            pl.BlockSpec((BLK, D), lambda i: (i, 0)),
            pl.BlockSpec((NC, BLK, D), lambda i: (0, i, 0)),
            pl.BlockSpec((D, D), lambda i: (0, 0)),
        ],
        out_specs=pl.BlockSpec((BLK, D), lambda i: (i, 0)),
        out_shape=jax.ShapeDtypeStruct((N, D), jnp.float32),
    )(out1, partials, W2)


def kernel(shape_features, edge_index, W1, b1, W2, b2):
    edge_i32 = edge_index.astype(jnp.int32)
    src, dst = edge_i32[0], edge_i32[1]
    zeros_tile = jnp.zeros((ROWS_PER_TILE, D), jnp.float32)
    out1 = _tc_out1(shape_features, W1, (b1 + b2).reshape(1, D))
    partials = _sc_aggregate(shape_features, src, dst, zeros_tile)
    return _tc_combine(out1, partials, W2)


# speedup vs baseline: 1.0056x; 1.0056x over previous
import functools

import jax
import jax.numpy as jnp
from jax import lax
from jax.experimental import pallas as pl
from jax.experimental.pallas import tpu as pltpu
from jax.experimental.pallas import tpu_sc as plsc

N = 10000
E = 320000
D = 128

NC = 2
NS = 16
NW = NC * NS
E_PER_W = E // NW
CHUNK = 80
N_CHUNKS = E_PER_W // CHUNK
RING = 3
AHEAD = 2
N_MAIN = N_CHUNKS - (N_CHUNKS % RING)
N_OUTER = N_MAIN // RING
N_PAD = 10240
ROWS_PER_TILE = N_PAD // NS


def _sc_aggregate(xb, src, dst, zeros_tile):
    mesh = plsc.VectorSubcoreMesh(core_axis_name="c", subcore_axis_name="s")

    @functools.partial(
        pl.kernel,
        out_type=jax.ShapeDtypeStruct((NC, N_PAD, D), jnp.float32),
        mesh=mesh,
        scratch_types=[
            pltpu.VMEM_SHARED((N_PAD, D), jnp.float32),
            pltpu.VMEM((E_PER_W,), jnp.int32),
            pltpu.VMEM((RING, CHUNK), jnp.int32),
            pltpu.VMEM((RING, CHUNK, D), jnp.float32),
            pltpu.SemaphoreType.DMA((RING,)),
            pltpu.SemaphoreType.DMA((RING,)),
            pltpu.SemaphoreType.DMA((RING,)),
        ],
    )
    def k(x_hbm, src_hbm, dst_hbm, z_hbm, out_hbm,
          acc, src_v, dst_v, rows_v, gsem, ssem, dsem):
        c = lax.axis_index("c")
        s = lax.axis_index("s")
        wid = s * NC + c
        base = wid * E_PER_W

        def goff(g):
            return pl.multiple_of(g * CHUNK, 8)

        def gather_start(g, b):
            idx = src_v.at[pl.ds(goff(g), CHUNK)]
            pltpu.async_copy(x_hbm.at[idx], rows_v.at[b], gsem.at[b])

        def gather_wait(b):
            idx = src_v.at[pl.ds(0, CHUNK)]
            pltpu.make_async_copy(x_hbm.at[idx], rows_v.at[b], gsem.at[b]).wait()

        def dst_start(g, b):
            pltpu.async_copy(dst_hbm.at[pl.ds(base + goff(g), CHUNK)],
                             dst_v.at[b], dsem.at[b])

        def dst_wait(b):
            pltpu.make_async_copy(dst_hbm.at[pl.ds(base, CHUNK)],
                                  dst_v.at[b], dsem.at[b]).wait()

        def scatter_start(b):
            pltpu.async_copy(rows_v.at[b], acc.at[dst_v.at[b]], ssem.at[b],
                             add=True)

        def scatter_wait(b):
            pltpu.make_async_copy(rows_v.at[b], acc.at[dst_v.at[b]],
                                  ssem.at[b]).wait()

        pltpu.sync_copy(src_hbm.at[pl.ds(base, E_PER_W)], src_v)
        for g in range(AHEAD):
            dst_start(g, g)
            gather_start(g, g)
        pltpu.sync_copy(z_hbm, acc.at[pl.ds(s * ROWS_PER_TILE, ROWS_PER_TILE)])
        plsc.subcore_barrier()

        def step(g, b):
            b2 = (b + AHEAD) % RING
            gather_wait(b)
            dst_wait(b)
            scatter_start(b)

            @pl.when(g >= RING - AHEAD)
            def _():
                scatter_wait(b2)

            @pl.when(g + AHEAD < N_CHUNKS)
            def _():
                dst_start(g + AHEAD, b2)
                gather_start(g + AHEAD, b2)

        def body(i, _):
            for b in range(RING):
                step(i * RING + b, b)
            return ()

        lax.fori_loop(0, N_OUTER, body, ())

        for g in range(N_MAIN, N_CHUNKS):
            step(g, g % RING)

        for g in range(N_CHUNKS - (RING - AHEAD), N_CHUNKS):
            scatter_wait(g % RING)

        plsc.subcore_barrier()

        pltpu.sync_copy(
            acc.at[pl.ds(s * ROWS_PER_TILE, ROWS_PER_TILE)],
            out_hbm.at[c, pl.ds(s * ROWS_PER_TILE, ROWS_PER_TILE)],
        )

    return k(xb, src, dst, zeros_tile)


def _tc_epilogue(x, partials, W1, W2, b):
    BLK = 400

    def body(x_ref, p_ref, w1_ref, w2_ref, b_ref, o_ref):
        dn = (((1,), (1,)), ((), ()))
        agg = p_ref[0] + p_ref[1]
        o_ref[...] = (
            lax.dot_general(x_ref[...], w1_ref[...], dn,
                            preferred_element_type=jnp.float32)
            + lax.dot_general(agg, w2_ref[...], dn,
                              preferred_element_type=jnp.float32)
            + b_ref[...]
        )

    return pl.pallas_call(
        body,
        grid=(N // BLK,),
        in_specs=[
            pl.BlockSpec((BLK, D), lambda i: (i, 0)),
            pl.BlockSpec((NC, BLK, D), lambda i: (0, i, 0)),
            pl.BlockSpec((D, D), lambda i: (0, 0)),
            pl.BlockSpec((D, D), lambda i: (0, 0)),
            pl.BlockSpec((1, D), lambda i: (0, 0)),
        ],
        out_specs=pl.BlockSpec((BLK, D), lambda i: (i, 0)),
        out_shape=jax.ShapeDtypeStruct((N, D), jnp.float32),
    )(x, partials, W1, W2, b)


def kernel(shape_features, edge_index, W1, b1, W2, b2):
    edge_i32 = edge_index.astype(jnp.int32)
    src, dst = edge_i32[0], edge_i32[1]
    zeros_tile = jnp.zeros((ROWS_PER_TILE, D), jnp.float32)
    partials = _sc_aggregate(shape_features, src, dst, zeros_tile)
    return _tc_epilogue(shape_features, partials, W1, W2,
                        (b1 + b2).reshape(1, D))
